# baseline (speedup 1.0000x reference)
import functools

import jax
import jax.numpy as jnp
from jax import lax
from jax.experimental import pallas as pl
from jax.experimental.pallas import tpu as pltpu
from jax.experimental.pallas import tpu_sc as plsc

_VOCAB = 1000000
_EMBED = 32
_BATCH = 16384
_PACK = 128 // _EMBED

_info = plsc.get_sparse_core_info()
_NC, _NS = _info.num_cores, _info.num_subcores
_NW = _NC * _NS
_B_PER_W = _BATCH // _NW
_CHUNK = 128
_NCHUNK = _B_PER_W // _CHUNK

_mesh = plsc.VectorSubcoreMesh(core_axis_name="c", subcore_axis_name="s")


@functools.partial(
    pl.kernel,
    out_type=jax.ShapeDtypeStruct((_BATCH, _EMBED), jnp.float32),
    mesh=_mesh,
    scratch_types=[
        pltpu.VMEM((_B_PER_W,), jnp.int32),
        pltpu.VMEM((_NCHUNK, _CHUNK), jnp.int32),
        pltpu.VMEM((2, _CHUNK, 128), jnp.float32),
        pltpu.VMEM((_B_PER_W, _EMBED), jnp.float32),
        pltpu.SemaphoreType.DMA,
        pltpu.SemaphoreType.DMA,
    ],
    compiler_params=pltpu.CompilerParams(needs_layout_passes=False),
)
def _gather_kernel(table_hbm, idx_hbm, out_hbm, idx_v, pidx_v, wide_v,
                   out_v, sem0, sem1):
    wid = lax.axis_index("s") * _NC + lax.axis_index("c")
    sems = (sem0, sem1)
    pltpu.sync_copy(idx_hbm.at[wid], idx_v)

    @pl.loop(0, _NCHUNK)
    def _pidx(j):
        @pl.loop(0, _CHUNK, step=16)
        def _pidx16(k):
            v = idx_v[pl.ds(j * _CHUNK + k, 16)]
            pidx_v[j, pl.ds(k, 16)] = lax.shift_right_logical(v, 2)

    def _fire(j):
        return pltpu.async_copy(
            table_hbm.at[pidx_v.at[j]], wide_v.at[j % 2], sems[j % 2])

    copies = [_fire(0)]
    for j in range(_NCHUNK):
        if j + 1 < _NCHUNK:
            copies.append(_fire(j + 1))
        copies[j].wait()

        @pl.loop(0, _CHUNK, step=16)
        def _extract(g):
            rows = jax.lax.iota(jnp.int32, 16) + g
            off = lax.shift_left(idx_v[pl.ds(j * _CHUNK + g, 16)] & 3, 5)
            orows = rows + j * _CHUNK
            for c in range(_EMBED):
                vals = plsc.load_gather(wide_v.at[j % 2], [rows, off + c])
                plsc.store_scatter(
                    out_v, [orows, jnp.full((16,), c, jnp.int32)], vals)

    pltpu.sync_copy(out_v, out_hbm.at[pl.ds(wid * _B_PER_W, _B_PER_W)])


def kernel(x, W_in):
    table = W_in.reshape(_VOCAB // _PACK, 128)
    idx2 = x.astype(jnp.int32).reshape(_NW, _B_PER_W)
    return _gather_kernel(table, idx2)


# speedup vs baseline: 4.2696x; 4.2696x over previous
import functools

import jax
import jax.numpy as jnp
from jax import lax
from jax.experimental import pallas as pl
from jax.experimental.pallas import tpu as pltpu
from jax.experimental.pallas import tpu_sc as plsc

_VOCAB = 1000000
_EMBED = 32
_BATCH = 16384

_info = plsc.get_sparse_core_info()
_NC, _NS = _info.num_cores, _info.num_subcores
_NW = _NC * _NS
_GL = 512
_NG = 61
_L_PER_W = _GL * _NG
_SIDE_LO = _L_PER_W * _NW + _GL
_NSIDE = _VOCAB - _SIDE_LO
_SEL_CAP = _BATCH + 16
_MCAP_VEC = 256
_ROWBUF = 272

_mesh = plsc.VectorSubcoreMesh(core_axis_name="c", subcore_axis_name="s")


@functools.partial(
    pl.kernel,
    out_type=jax.ShapeDtypeStruct((_BATCH * _EMBED,), jnp.float32),
    mesh=_mesh,
    scratch_types=[
        pltpu.VMEM((1024,), jnp.int32),
        pltpu.VMEM((_SEL_CAP,), jnp.int32),
        pltpu.VMEM((_MCAP_VEC * 16 + 16,), jnp.int32),
        pltpu.VMEM((2, _EMBED, _GL), jnp.float32),
        pltpu.VMEM((_ROWBUF, _EMBED), jnp.float32),
        pltpu.VMEM((_NSIDE * _EMBED,), jnp.float32),
        pltpu.SemaphoreType.DMA,
        pltpu.SemaphoreType.DMA,
        pltpu.SemaphoreType.DMA,
    ],
    compiler_params=pltpu.CompilerParams(needs_layout_passes=False),
)
def _scan_kernel(table_hbm, idx_hbm, side_hbm, out_hbm, xc_v, selp_v,
                 m_v, bufs, rowbuf, side_v, sem_g0, sem_g1, sem_row):
    i32 = jnp.int32
    wid = lax.axis_index("s") * _NC + lax.axis_index("c")
    lane_lo = wid * _L_PER_W
    lane_hi = jnp.where(wid == _NW - 1, _VOCAB, lane_lo + _L_PER_W)
    c16 = lax.iota(i32, 16)
    sems_g = (sem_g0, sem_g1)

    def fire_group(g, b):
        lo = pl.multiple_of(lane_lo + g * _GL, 128)
        pltpu.async_copy(
            table_hbm.at[:, pl.ds(lo, _GL)], bufs.at[b], sems_g[b])

    def wait_group(b):
        pltpu.make_async_copy(
            table_hbm.at[:, pl.ds(0, _GL)], bufs.at[b], sems_g[b]).wait()

    def drain_rows(n):
        def _w(_, c):
            pltpu.make_async_copy(
                out_hbm.at[pl.ds(0, _EMBED)], rowbuf.at[0], sem_row).wait()
            return c
        lax.fori_loop(0, n, _w, 0)

    fire_group(0, 0)
    fire_group(1, 1)

    def sel_chunk(ch, nsel):
        pltpu.sync_copy(idx_hbm.at[pl.ds(ch * 1024, 1024)], xc_v)

        def sel_vec(j, nsel):
            v = xc_v[pl.ds(j * 16, 16)]
            msk = (v >= lane_lo) & (v < lane_hi)
            kv = ch * 1024 + j * 16 + c16
            packed = lax.shift_left(v - lane_lo, 14) | kv
            plsc.store_compressed(selp_v.at[pl.ds(nsel, 16)], packed,
                                  mask=msk)
            return nsel + plsc.all_reduce_population_count(msk)[0]

        return lax.fori_loop(0, 64, sel_vec, nsel)

    nsel = lax.fori_loop(0, _BATCH // 1024, sel_chunk, jnp.asarray(0, i32))
    nselvec = (nsel + 15) // 16
    nrounds = (nselvec + _MCAP_VEC - 1) // _MCAP_VEC

    @pl.when(wid == _NW - 1)
    def _():
        pltpu.sync_copy(side_hbm, side_v)

    def match_range(grel_lo, grel_hi, vec0):
        def mbody(j, nm):
            v = selp_v[pl.ds(j * 16, 16)]
            xr = lax.shift_right_logical(v, 14)
            lane_id = j * 16 + c16
            msk = (xr >= grel_lo) & (xr < grel_hi) & (lane_id < nsel)
            packed = lax.shift_left(xr - grel_lo, 16) | (v & 0x3FFF)
            plsc.store_compressed(m_v.at[pl.ds(nm, 16)], packed, mask=msk)
            return nm + plsc.all_reduce_population_count(msk)[0]

        return lax.fori_loop(vec0, jnp.minimum(nselvec, vec0 + _MCAP_VEC),
                             mbody, jnp.asarray(0, i32))

    def extract(nm, w, gather_row):
        def ebody(i, w):
            w = lax.cond(w + 16 > _ROWBUF - 16,
                         lambda ww: (drain_rows(ww), jnp.asarray(0, i32))[1],
                         lambda ww: ww, w)
            pv = m_v[pl.ds(i * 16, 16)]
            for j in range(16):
                @pl.when(i * 16 + j < nm)
                def _():
                    p = pv[j]
                    joff = lax.shift_right_logical(p, 16)
                    ks = p & 0xFFFF
                    v0, v1 = gather_row(joff)
                    rowbuf[w + j, pl.ds(0, 16)] = v0
                    rowbuf[w + j, pl.ds(16, 16)] = v1
                    pltpu.async_copy(
                        rowbuf.at[w + j],
                        out_hbm.at[pl.ds(
                            pl.multiple_of(ks * _EMBED, 32), _EMBED)],
                        sem_row)
            return w + jnp.minimum(nm - i * 16, 16)

        return lax.fori_loop(0, (nm + 15) // 16, ebody, w)

    def gather_from_buf(b):
        def _g(joff):
            jv = jnp.full((16,), joff, i32)
            return (plsc.load_gather(bufs.at[b], [c16, jv]),
                    plsc.load_gather(bufs.at[b], [c16 + 16, jv]))
        return _g

    def gather_from_side(joff):
        base = joff * _EMBED + c16
        return (plsc.load_gather(side_v, [base]),
                plsc.load_gather(side_v, [base + 16]))

    def step(g, b, w):
        wait_group(b)
        grel = g * _GL

        def rbody(r, w):
            nm = match_range(grel, grel + _GL, r * _MCAP_VEC)
            return extract(nm, w, gather_from_buf(b))

        return lax.fori_loop(0, nrounds, rbody, w)

    def pair(k2, w):
        g = k2 * 2
        w = step(g, 0, w)
        fire_group(g + 2, 0)
        w = step(g + 1, 1, w)

        @pl.when(g + 3 < _NG)
        def _():
            fire_group(g + 3, 1)
        return w

    w = lax.fori_loop(0, (_NG - 1) // 2, pair, jnp.asarray(0, i32))
    fire_group(_NG, 1)
    w = step(_NG - 1, 0, w)
    w = step(_NG, 1, w)

    def sround(r, w):
        nm = match_range(jnp.asarray(_L_PER_W + _GL, i32),
                         jnp.asarray(_L_PER_W + _GL + _NSIDE, i32),
                         r * _MCAP_VEC)
        return extract(nm, w, gather_from_side)

    w = lax.fori_loop(0, nrounds, sround, w)

    drain_rows(w)


def kernel(x, W_in):
    table_t = jnp.swapaxes(W_in, 0, 1)
    side = W_in[_SIDE_LO:].reshape(-1)
    out_flat = _scan_kernel(table_t, x.astype(jnp.int32), side)
    return out_flat.reshape(_BATCH, _EMBED)


# speedup vs baseline: 4.5270x; 1.0603x over previous
import functools

import jax
import jax.numpy as jnp
from jax import lax
from jax.experimental import pallas as pl
from jax.experimental.pallas import tpu as pltpu
from jax.experimental.pallas import tpu_sc as plsc

_VOCAB = 1000000
_EMBED = 32
_BATCH = 16384

_info = plsc.get_sparse_core_info()
_NC, _NS = _info.num_cores, _info.num_subcores
_NW = _NC * _NS
_GL = 512
_NG = 61
_L_PER_W = _GL * _NG
_SIDE_LO = _L_PER_W * _NW + _GL
_NSIDE = _VOCAB - _SIDE_LO
_SEL_CAP = _BATCH + 16
_MCAP_VEC = 128
_ROWBUF = 144

_mesh = plsc.VectorSubcoreMesh(core_axis_name="c", subcore_axis_name="s")


@functools.partial(
    pl.kernel,
    out_type=jax.ShapeDtypeStruct((_BATCH * _EMBED,), jnp.float32),
    mesh=_mesh,
    scratch_types=[
        pltpu.VMEM((512,), jnp.int32),
        pltpu.VMEM((_SEL_CAP,), jnp.int32),
        pltpu.VMEM((_MCAP_VEC * 16 + 16,), jnp.int32),
        pltpu.VMEM((3, _EMBED, _GL), jnp.float32),
        pltpu.VMEM((_ROWBUF, _EMBED), jnp.float32),
        pltpu.VMEM((_NSIDE * _EMBED,), jnp.float32),
        pltpu.SemaphoreType.DMA,
        pltpu.SemaphoreType.DMA,
        pltpu.SemaphoreType.DMA,
        pltpu.SemaphoreType.DMA,
    ],
    compiler_params=pltpu.CompilerParams(needs_layout_passes=False),
)
def _scan_kernel(table_hbm, idx_hbm, side_hbm, out_hbm, xc_v, selp_v,
                 m_v, bufs, rowbuf, side_v, sem_g0, sem_g1, sem_g2, sem_row):
    i32 = jnp.int32
    wid = lax.axis_index("s") * _NC + lax.axis_index("c")
    lane_lo = wid * _L_PER_W
    lane_hi = jnp.where(wid == _NW - 1, _VOCAB, lane_lo + _L_PER_W)
    c16 = lax.iota(i32, 16)
    sems_g = (sem_g0, sem_g1, sem_g2)

    def fire_group(g, b):
        lo = pl.multiple_of(lane_lo + g * _GL, 128)
        pltpu.async_copy(
            table_hbm.at[:, pl.ds(lo, _GL)], bufs.at[b], sems_g[b])

    def wait_group(b):
        pltpu.make_async_copy(
            table_hbm.at[:, pl.ds(0, _GL)], bufs.at[b], sems_g[b]).wait()

    def drain_rows(n):
        def _w(_, c):
            pltpu.make_async_copy(
                out_hbm.at[pl.ds(0, _EMBED)], rowbuf.at[0], sem_row).wait()
            return c
        lax.fori_loop(0, n, _w, 0)

    fire_group(0, 0)
    fire_group(1, 1)
    fire_group(2, 2)

    def sel_chunk(ch, nsel):
        pltpu.sync_copy(idx_hbm.at[pl.ds(ch * 512, 512)], xc_v)

        def sel_vec(j, nsel):
            v = xc_v[pl.ds(j * 16, 16)]
            msk = (v >= lane_lo) & (v < lane_hi)
            kv = ch * 512 + j * 16 + c16
            packed = lax.shift_left(v - lane_lo, 14) | kv
            plsc.store_compressed(selp_v.at[pl.ds(nsel, 16)], packed,
                                  mask=msk)
            return nsel + plsc.all_reduce_population_count(msk)[0]

        return lax.fori_loop(0, 32, sel_vec, nsel)

    nsel = lax.fori_loop(0, _BATCH // 512, sel_chunk, jnp.asarray(0, i32))
    selp_v[pl.ds(nsel, 16)] = jnp.full((16,), 0x7FFFFFFF, i32)
    nselvec = (nsel + 15) // 16
    nrounds = (nselvec + _MCAP_VEC - 1) // _MCAP_VEC

    @pl.when(wid == _NW - 1)
    def _():
        pltpu.sync_copy(side_hbm, side_v)

    def match_range(grel_lo, grel_hi, vec0):
        def mbody(j, nm):
            v = selp_v[pl.ds(j * 16, 16)]
            xr = lax.shift_right_logical(v, 14)
            msk = (xr >= grel_lo) & (xr < grel_hi)
            packed = lax.shift_left(xr - grel_lo, 16) | (v & 0x3FFF)
            plsc.store_compressed(m_v.at[pl.ds(nm, 16)], packed, mask=msk)
            return nm + plsc.all_reduce_population_count(msk)[0]

        return lax.fori_loop(vec0, jnp.minimum(nselvec, vec0 + _MCAP_VEC),
                             mbody, jnp.asarray(0, i32))

    def extract(nm, w, gather_row):
        def ebody(i, w):
            w = lax.cond(w + 16 > _ROWBUF - 16,
                         lambda ww: (drain_rows(ww), jnp.asarray(0, i32))[1],
                         lambda ww: ww, w)
            pv = m_v[pl.ds(i * 16, 16)]
            for j in range(16):
                @pl.when(i * 16 + j < nm)
                def _():
                    p = pv[j]
                    joff = lax.shift_right_logical(p, 16)
                    ks = p & 0xFFFF
                    v0, v1 = gather_row(joff)
                    rowbuf[w + j, pl.ds(0, 16)] = v0
                    rowbuf[w + j, pl.ds(16, 16)] = v1
                    pltpu.async_copy(
                        rowbuf.at[w + j],
                        out_hbm.at[pl.ds(
                            pl.multiple_of(ks * _EMBED, 32), _EMBED)],
                        sem_row)
            return w + jnp.minimum(nm - i * 16, 16)

        return lax.fori_loop(0, (nm + 15) // 16, ebody, w)

    def gather_from_buf(b):
        def _g(joff):
            jv = jnp.full((16,), joff, i32)
            return (plsc.load_gather(bufs.at[b], [c16, jv]),
                    plsc.load_gather(bufs.at[b], [c16 + 16, jv]))
        return _g

    def gather_from_side(joff):
        base = joff * _EMBED + c16
        return (plsc.load_gather(side_v, [base]),
                plsc.load_gather(side_v, [base + 16]))

    def step(g, b, w):
        wait_group(b)
        grel = g * _GL

        def rbody(r, w):
            nm = match_range(grel, grel + _GL, r * _MCAP_VEC)
            return extract(nm, w, gather_from_buf(b))

        return lax.fori_loop(0, nrounds, rbody, w)

    def triple(k3, w):
        g = k3 * 3
        for b in range(3):
            w = step(g + b, b, w)

            @pl.when(g + b + 3 <= _NG)
            def _():
                fire_group(g + b + 3, b)
        return w

    w = lax.fori_loop(0, 20, triple, jnp.asarray(0, i32))
    w = step(_NG - 1, 0, w)
    w = step(_NG, 1, w)

    def sround(r, w):
        nm = match_range(jnp.asarray(_L_PER_W + _GL, i32),
                         jnp.asarray(_L_PER_W + _GL + _NSIDE, i32),
                         r * _MCAP_VEC)
        return extract(nm, w, gather_from_side)

    w = lax.fori_loop(0, nrounds, sround, w)

    drain_rows(w)


def kernel(x, W_in):
    table_t = jnp.swapaxes(W_in, 0, 1)
    side = W_in[_SIDE_LO:].reshape(-1)
    out_flat = _scan_kernel(table_t, x.astype(jnp.int32), side)
    return out_flat.reshape(_BATCH, _EMBED)
